# Initial kernel scaffold; baseline (speedup 1.0000x reference)
import jax
import jax.numpy as jnp
from jax.experimental import pallas as pl


def kernel(molec_feature_vectures, mo_neighbour_indices, mo_mol_id, mo_pair_id, V_n, wfn_pairs, wfn_pairs_mol_id, n_output, W):
    raise NotImplementedError("write your pallas kernel here")


# speedup vs baseline: 5.1815x; 5.1815x over previous
import functools

import jax
import jax.numpy as jnp
from jax import lax
from jax.experimental import pallas as pl
from jax.experimental.pallas import tpu as pltpu
from jax.experimental.pallas import tpu_sc as plsc

N_NODES = 10000
F = 128
NC, NS = 2, 16
NW = NC * NS
CH = 128
ACC_ROWS = 10240
DUMMY_ROW = N_NODES
ROWS_PER_TILE_OUT = ACC_ROWS // NS


def _sc_segment_accumulate(feat, src_p, dst_p, t_chunks):
    mesh = plsc.VectorSubcoreMesh(core_axis_name="c", subcore_axis_name="s")

    @functools.partial(
        pl.kernel,
        mesh=mesh,
        out_type=jax.ShapeDtypeStruct((NC, ACC_ROWS, F), jnp.float32),
        scratch_types=[
            pltpu.VMEM((t_chunks, CH), jnp.int32),
            pltpu.VMEM((t_chunks, CH), jnp.int32),
            pltpu.VMEM((CH, F), jnp.float32),
            pltpu.VMEM((CH, F), jnp.float32),
            pltpu.VMEM((16, F), jnp.float32),
            pltpu.VMEM_SHARED((ACC_ROWS, F), jnp.float32),
            pltpu.SemaphoreType.DMA,
            pltpu.SemaphoreType.DMA,
        ],
    )
    def k(feat_hbm, src_hbm, dst_hbm, out_hbm,
          src_v, dst_v, rows_a, rows_b, zero_v, acc, sem_a, sem_b):
        c = lax.axis_index("c")
        s = lax.axis_index("s")
        w = c * NS + s

        zf = jnp.zeros((16,), jnp.float32)
        for r in range(16):
            for g in range(F // 16):
                zero_v[r, pl.ds(g * 16, 16)] = zf

        n_zero_blocks = ACC_ROWS // (NS * 16)
        zbase = s * (ACC_ROWS // NS)

        def zbody(i, carry):
            pltpu.sync_copy(zero_v, acc.at[pl.ds(zbase + i * 16, 16)])
            return carry
        lax.fori_loop(0, n_zero_blocks, zbody, None)

        pltpu.sync_copy(src_hbm.at[w], src_v)
        pltpu.sync_copy(dst_hbm.at[w], dst_v)

        plsc.subcore_barrier()

        def body(j, carry):
            pltpu.async_copy(feat_hbm.at[dst_v.at[j]], rows_a, sem_a).wait()
            pltpu.sync_copy(rows_a, acc.at[src_v.at[j]], add=True)
            return carry
        lax.fori_loop(0, t_chunks, body, None)

        plsc.subcore_barrier()

        obase = s * ROWS_PER_TILE_OUT
        pltpu.sync_copy(acc.at[pl.ds(obase, ROWS_PER_TILE_OUT)],
                        out_hbm.at[c, pl.ds(obase, ROWS_PER_TILE_OUT)])

    return k(feat, src_p, dst_p)


def _tc_combine(feat, w_mat, gp):
    def body(feat_ref, w_ref, gp_ref, out_ref):
        m = w_ref[...] + w_ref[...].T
        h = jnp.dot(feat_ref[...], m, preferred_element_type=jnp.float32)
        g = gp_ref[0, :N_NODES] + gp_ref[1, :N_NODES]
        out_ref[...] = jnp.sum(h * g, axis=1)

    return pl.pallas_call(
        body,
        out_shape=jax.ShapeDtypeStruct((N_NODES,), jnp.float32),
    )(feat, w_mat, gp)


def kernel(molec_feature_vectures, mo_neighbour_indices, mo_mol_id, mo_pair_id,
           V_n, wfn_pairs, wfn_pairs_mol_id, n_output, W):
    feat = molec_feature_vectures
    src = mo_neighbour_indices[0]
    dst = mo_neighbour_indices[1]
    e = src.shape[0]
    per_chunk_all = NW * CH
    t_chunks = -(-e // per_chunk_all)
    pad = t_chunks * per_chunk_all - e
    src_p = jnp.concatenate(
        [src, jnp.full((pad,), DUMMY_ROW, jnp.int32)]).reshape(NW, t_chunks, CH)
    dst_p = jnp.concatenate(
        [dst, jnp.zeros((pad,), jnp.int32)]).reshape(NW, t_chunks, CH)
    gp = _sc_segment_accumulate(feat, src_p, dst_p, t_chunks)
    return _tc_combine(feat, W, gp)
